# baseline (speedup 1.0000x reference)
import functools

import jax
import jax.numpy as jnp
from jax import lax
from jax.experimental import pallas as pl
from jax.experimental.pallas import tpu as pltpu
from jax.experimental.pallas import tpu_sc as plsc

NUM_ENTS = 1000000
ENT_DIM = 64
NEG_RATIO = 7
MARGIN = 1.0
N = 32768

NC = 2
NS = 16
NW = NC * NS
LANES = 16
PER_W = N // NW
CHUNK = 128
GROUPS_PER_CHUNK = CHUNK // (NEG_RATIO + 1)
ROW = 2 * ENT_DIM


def _rsqrt_nr(s):
    i = lax.bitcast_convert_type(s, jnp.int32)
    i = jnp.int32(0x5F3759DF) - (i >> 1)
    y = lax.bitcast_convert_type(i, jnp.float32)
    half_s = s * 0.5
    for _ in range(3):
        y = y * (1.5 - half_s * y * y)
    return y


def _make_sc_kernel():
    mesh = plsc.VectorSubcoreMesh(core_axis_name="c", subcore_axis_name="s")

    @functools.partial(
        pl.kernel,
        mesh=mesh,
        compiler_params=pltpu.CompilerParams(needs_layout_passes=False),
        out_type=jax.ShapeDtypeStruct((NW, LANES), jnp.float32),
        scratch_types=[
            pltpu.VMEM((CHUNK,), jnp.int32),
            pltpu.VMEM((CHUNK,), jnp.int32),
            pltpu.VMEM((CHUNK,), jnp.int32),
            pltpu.VMEM((CHUNK,), jnp.int32),
            pltpu.VMEM((CHUNK,), jnp.int32),
            pltpu.VMEM((CHUNK,), jnp.int32),
            pltpu.VMEM((CHUNK, ROW), jnp.float32),
            pltpu.VMEM((CHUNK, ROW), jnp.float32),
            pltpu.VMEM((CHUNK, ROW), jnp.float32),
            pltpu.VMEM((LANES,), jnp.float32),
            pltpu.SemaphoreType.DMA,
        ],
    )
    def sc_kernel(bh_hbm, br_hbm, bt_hbm, ent_hbm, rel_hbm, out_hbm,
                  bid_h, bid_r, bid_t, idx_h, idx_r, idx_t,
                  rows_h, rows_r, rows_t, part_v, sem):
        wid = lax.axis_index("s") * NC + lax.axis_index("c")
        loss = jnp.float32(0.0)

        for c in range(PER_W // CHUNK):
            base = wid * PER_W + c * CHUNK
            pltpu.sync_copy(bh_hbm.at[pl.ds(base, CHUNK)], bid_h)
            pltpu.sync_copy(br_hbm.at[pl.ds(base, CHUNK)], bid_r)
            pltpu.sync_copy(bt_hbm.at[pl.ds(base, CHUNK)], bid_t)
            for v in range(CHUNK // LANES):
                sl = pl.ds(v * LANES, LANES)
                idx_h[sl] = bid_h[sl] >> 1
                idx_r[sl] = bid_r[sl] >> 1
                idx_t[sl] = bid_t[sl] >> 1
            cp_h = pltpu.async_copy(ent_hbm.at[idx_h], rows_h, sem)
            cp_r = pltpu.async_copy(rel_hbm.at[idx_r], rows_r, sem)
            cp_t = pltpu.async_copy(ent_hbm.at[idx_t], rows_t, sem)
            cp_h.wait()
            cp_r.wait()
            cp_t.wait()

            def group_body(g, loss_acc):
                gb = g * (NEG_RATIO + 1)
                vh = bid_h[pl.ds(gb, LANES)]
                vr = bid_r[pl.ds(gb, LANES)]
                vt = bid_t[pl.ds(gb, LANES)]
                scores = []
                for j in range(NEG_RATIO + 1):
                    row = gb + j
                    oh = (vh[j] & 1) * ENT_DIM
                    orr = (vr[j] & 1) * ENT_DIM
                    ot = (vt[j] & 1) * ENT_DIM
                    h = [rows_h[row, pl.ds(oh + k * LANES, LANES)] for k in range(ENT_DIM // LANES)]
                    r = [rows_r[row, pl.ds(orr + k * LANES, LANES)] for k in range(ENT_DIM // LANES)]
                    t = [rows_t[row, pl.ds(ot + k * LANES, LANES)] for k in range(ENT_DIM // LANES)]
                    ssh = h[0] * h[0] + h[1] * h[1] + h[2] * h[2] + h[3] * h[3]
                    ssr = r[0] * r[0] + r[1] * r[1] + r[2] * r[2] + r[3] * r[3]
                    sst = t[0] * t[0] + t[1] * t[1] + t[2] * t[2] + t[3] * t[3]
                    inh = _rsqrt_nr(jnp.full((LANES,), jnp.maximum(jnp.sum(ssh), 1e-24)))
                    inr = _rsqrt_nr(jnp.full((LANES,), jnp.maximum(jnp.sum(ssr), 1e-24)))
                    int_ = _rsqrt_nr(jnp.full((LANES,), jnp.maximum(jnp.sum(sst), 1e-24)))
                    acc = jnp.abs(h[0] * inh + r[0] * inr - t[0] * int_)
                    for k in range(1, ENT_DIM // LANES):
                        acc = acc + jnp.abs(h[k] * inh + r[k] * inr - t[k] * int_)
                    scores.append(jnp.sum(acc))
                neg = scores[1]
                for j in range(2, NEG_RATIO + 1):
                    neg = neg + scores[j]
                term = scores[0] - neg * jnp.float32(1.0 / NEG_RATIO) + jnp.float32(MARGIN)
                return loss_acc + jnp.maximum(term, 0.0)

            loss = lax.fori_loop(0, GROUPS_PER_CHUNK, group_body, loss)

        part_v[...] = jnp.full((LANES,), loss)
        pltpu.sync_copy(part_v, out_hbm.at[wid])

    return sc_kernel


_SC_KERNEL = _make_sc_kernel()


def kernel(batch_h, batch_r, batch_t, batch_y, ent_table, rel_table):
    del batch_y
    ent2 = ent_table.reshape(NUM_ENTS // 2, ROW)
    rel2 = rel_table.reshape(-1, ROW)
    parts = _SC_KERNEL(batch_h.astype(jnp.int32), batch_r.astype(jnp.int32),
                       batch_t.astype(jnp.int32), ent2, rel2)
    return jnp.sum(parts[:, 0])


# speedup vs baseline: 1.7572x; 1.7572x over previous
import functools

import jax
import jax.numpy as jnp
from jax import lax
from jax.experimental import pallas as pl
from jax.experimental.pallas import tpu as pltpu
from jax.experimental.pallas import tpu_sc as plsc

NUM_ENTS = 1000000
ENT_DIM = 64
NEG_RATIO = 7
MARGIN = 1.0
N = 32768

NC = 2
NS = 16
NW = NC * NS
LANES = 16
PER_W = N // NW
CHUNK = 128
NCHUNK = PER_W // CHUNK
NPAIR = NCHUNK // 2
GROUPS_PER_CHUNK = CHUNK // (NEG_RATIO + 1)
KV = ENT_DIM // LANES


def _rsqrt_nr(s):
    i = lax.bitcast_convert_type(s, jnp.int32)
    i = jnp.int32(0x5F3759DF) - (i >> 1)
    y = lax.bitcast_convert_type(i, jnp.float32)
    half_s = s * 0.5
    for _ in range(3):
        y = y * (1.5 - half_s * y * y)
    return y


def _inv_norm(vregs):
    ss = vregs[0] * vregs[0]
    for k in range(1, len(vregs)):
        ss = ss + vregs[k] * vregs[k]
    return _rsqrt_nr(jnp.full((LANES,), jnp.maximum(jnp.sum(ss), 1e-24)))


def _make_rel_kernel():
    mesh = plsc.VectorSubcoreMesh(core_axis_name="c", subcore_axis_name="s")

    @functools.partial(
        pl.kernel,
        mesh=mesh,
        compiler_params=pltpu.CompilerParams(needs_layout_passes=False),
        out_type=jax.ShapeDtypeStruct((N, ENT_DIM), jnp.float32),
        scratch_types=[
            pltpu.VMEM((PER_W,), jnp.int32),
            pltpu.VMEM((CHUNK, ENT_DIM), jnp.float32),
            pltpu.VMEM((CHUNK, ENT_DIM), jnp.float32),
            pltpu.VMEM((CHUNK, ENT_DIM), jnp.float32),
            pltpu.SemaphoreType.DMA,
            pltpu.SemaphoreType.DMA,
        ],
    )
    def rel_kernel(br_hbm, rel_hbm, rhat_hbm, bid_r, rw0, rw1, nrm, sem0, sem1):
        wid = lax.axis_index("s") * NC + lax.axis_index("c")
        base = wid * PER_W
        pltpu.sync_copy(br_hbm.at[pl.ds(base, PER_W)], bid_r)
        bufs = [(rw0, sem0), (rw1, sem1)]

        def fire(c, parity):
            rw, sem = bufs[parity]
            def vblock(vb, _):
                vr = bid_r[pl.ds(c * CHUNK + vb * LANES, LANES)]
                for j in range(LANES):
                    pltpu.async_copy(rel_hbm.at[vr[j]], rw.at[vb * LANES + j], sem)
                return 0
            lax.fori_loop(0, CHUNK // LANES, vblock, 0)

        def drain(parity):
            rw, sem = bufs[parity]
            pltpu.make_async_copy(rel_hbm.at[pl.ds(0, CHUNK)], rw, sem).wait()

        def process(c, parity):
            rw, _ = bufs[parity]
            def row_body(i, _):
                r = [rw[i, pl.ds(k * LANES, LANES)] for k in range(KV)]
                inr = _inv_norm(r)
                for k in range(KV):
                    nrm[i, pl.ds(k * LANES, LANES)] = r[k] * inr
                return 0
            lax.fori_loop(0, CHUNK, row_body, 0)
            pltpu.sync_copy(nrm, rhat_hbm.at[pl.ds(base + c * CHUNK, CHUNK)])

        fire(0, 0)
        fire(1, 1)

        def pair_body(p, _):
            c0 = 2 * p
            drain(0)
            process(c0, 0)
            fire(jnp.minimum(c0 + 2, NCHUNK - 1), 0)
            drain(1)
            process(c0 + 1, 1)
            fire(jnp.minimum(c0 + 3, NCHUNK - 1), 1)
            return 0

        lax.fori_loop(0, NPAIR, pair_body, 0)
        drain(0)
        drain(1)

    return rel_kernel


def _make_main_kernel():
    mesh = plsc.VectorSubcoreMesh(core_axis_name="c", subcore_axis_name="s")

    @functools.partial(
        pl.kernel,
        mesh=mesh,
        compiler_params=pltpu.CompilerParams(needs_layout_passes=False),
        out_type=jax.ShapeDtypeStruct((NW, LANES), jnp.float32),
        scratch_types=[
            pltpu.VMEM((PER_W,), jnp.int32),
            pltpu.VMEM((PER_W,), jnp.int32),
            pltpu.VMEM((CHUNK, ENT_DIM), jnp.float32),
            pltpu.VMEM((CHUNK, ENT_DIM), jnp.float32),
            pltpu.VMEM((CHUNK, ENT_DIM), jnp.float32),
            pltpu.VMEM((CHUNK, ENT_DIM), jnp.float32),
            pltpu.VMEM((CHUNK, ENT_DIM), jnp.float32),
            pltpu.VMEM((CHUNK, ENT_DIM), jnp.float32),
            pltpu.VMEM((LANES,), jnp.float32),
            pltpu.SemaphoreType.DMA,
            pltpu.SemaphoreType.DMA,
        ],
    )
    def sc_kernel(bh_hbm, bt_hbm, ent_hbm, rhat_hbm, out_hbm,
                  bid_h, bid_t,
                  rh0, rr0, rt0, rh1, rr1, rt1, part_v, sem0, sem1):
        wid = lax.axis_index("s") * NC + lax.axis_index("c")
        base = wid * PER_W
        pltpu.sync_copy(bh_hbm.at[pl.ds(base, PER_W)], bid_h)
        pltpu.sync_copy(bt_hbm.at[pl.ds(base, PER_W)], bid_t)

        bufs = [(rh0, rr0, rt0, sem0), (rh1, rr1, rt1, sem1)]

        def fire(c, parity):
            rh, rr, rt, sem = bufs[parity]
            pltpu.async_copy(rhat_hbm.at[pl.ds(base + c * CHUNK, CHUNK)], rr, sem)
            def vblock(vb, _):
                cb = c * CHUNK + vb * LANES
                vh = bid_h[pl.ds(cb, LANES)]
                vt = bid_t[pl.ds(cb, LANES)]
                for j in range(LANES):
                    row = vb * LANES + j
                    pltpu.async_copy(ent_hbm.at[vh[j]], rh.at[row], sem)
                    pltpu.async_copy(ent_hbm.at[vt[j]], rt.at[row], sem)
                return 0
            lax.fori_loop(0, CHUNK // LANES, vblock, 0)

        def drain(parity):
            rh, rr, rt, sem = bufs[parity]
            pltpu.make_async_copy(ent_hbm.at[pl.ds(0, CHUNK)], rh, sem).wait()
            pltpu.make_async_copy(ent_hbm.at[pl.ds(0, CHUNK)], rr, sem).wait()
            pltpu.make_async_copy(ent_hbm.at[pl.ds(0, CHUNK)], rt, sem).wait()

        def compute(parity, loss):
            rh, rr, rt, _ = bufs[parity]

            def group_body(g, loss_acc):
                gb = g * (NEG_RATIO + 1)
                gacc = jnp.zeros((LANES,), jnp.float32)
                for j in range(NEG_RATIO + 1):
                    row = gb + j
                    w = jnp.float32(1.0 if j == 0 else -1.0 / NEG_RATIO)
                    h = [rh[row, pl.ds(k * LANES, LANES)] for k in range(KV)]
                    r = [rr[row, pl.ds(k * LANES, LANES)] for k in range(KV)]
                    t = [rt[row, pl.ds(k * LANES, LANES)] for k in range(KV)]
                    inh = _inv_norm(h)
                    int_ = _inv_norm(t)
                    acc = jnp.abs(h[0] * inh + r[0] - t[0] * int_)
                    for k in range(1, KV):
                        acc = acc + jnp.abs(h[k] * inh + r[k] - t[k] * int_)
                    gacc = gacc + w * acc
                term = jnp.sum(gacc) + jnp.float32(MARGIN)
                return loss_acc + jnp.maximum(term, 0.0)

            return lax.fori_loop(0, GROUPS_PER_CHUNK, group_body, loss)

        fire(0, 0)
        fire(1, 1)

        def pair_body(p, loss):
            c0 = 2 * p
            drain(0)
            loss = compute(0, loss)
            fire(jnp.minimum(c0 + 2, NCHUNK - 1), 0)
            drain(1)
            loss = compute(1, loss)
            fire(jnp.minimum(c0 + 3, NCHUNK - 1), 1)
            return loss

        loss = lax.fori_loop(0, NPAIR, pair_body, jnp.float32(0.0))
        drain(0)
        drain(1)

        part_v[...] = jnp.full((LANES,), loss)
        pltpu.sync_copy(part_v, out_hbm.at[wid])

    return sc_kernel


_REL_KERNEL = _make_rel_kernel()
_MAIN_KERNEL = _make_main_kernel()


def kernel(batch_h, batch_r, batch_t, batch_y, ent_table, rel_table):
    del batch_y
    rhat = _REL_KERNEL(batch_r.astype(jnp.int32), rel_table)
    parts = _MAIN_KERNEL(batch_h.astype(jnp.int32), batch_t.astype(jnp.int32),
                         ent_table, rhat)
    return jnp.sum(parts[:, 0])


# speedup vs baseline: 2.1080x; 1.1996x over previous
import functools

import jax
import jax.numpy as jnp
from jax import lax
from jax.experimental import pallas as pl
from jax.experimental.pallas import tpu as pltpu
from jax.experimental.pallas import tpu_sc as plsc

NUM_ENTS = 1000000
ENT_DIM = 64
NEG_RATIO = 7
MARGIN = 1.0
N = 32768

NC = 2
NS = 16
NW = NC * NS
LANES = 16
PER_W = N // NW
CHUNK = 128
NCHUNK = PER_W // CHUNK
NPAIR = NCHUNK // 2
GROUPS_PER_CHUNK = CHUNK // (NEG_RATIO + 1)
KV = ENT_DIM // LANES


def _rsqrt_nr(s):
    i = lax.bitcast_convert_type(s, jnp.int32)
    i = jnp.int32(0x5F3759DF) - (i >> 1)
    y = lax.bitcast_convert_type(i, jnp.float32)
    half_s = s * 0.5
    for _ in range(3):
        y = y * (1.5 - half_s * y * y)
    return y


def _inv_norm(vregs):
    ss = vregs[0] * vregs[0]
    for k in range(1, len(vregs)):
        ss = ss + vregs[k] * vregs[k]
    return _rsqrt_nr(jnp.full((LANES,), jnp.maximum(jnp.sum(ss), 1e-24)))


def _make_tc_transpose():
    lanes_per_step = 8192

    def body(x_ref, o_ref):
        o_ref[...] = x_ref[...].T

    return pl.pallas_call(
        body,
        grid=(pl.cdiv(NUM_ENTS, lanes_per_step),),
        in_specs=[pl.BlockSpec((ENT_DIM, lanes_per_step), lambda i: (0, i))],
        out_specs=pl.BlockSpec((lanes_per_step, ENT_DIM), lambda i: (i, 0)),
        out_shape=jax.ShapeDtypeStruct((NUM_ENTS, ENT_DIM), jnp.float32),
    )


def _make_rel_kernel():
    mesh = plsc.VectorSubcoreMesh(core_axis_name="c", subcore_axis_name="s")

    @functools.partial(
        pl.kernel,
        mesh=mesh,
        compiler_params=pltpu.CompilerParams(needs_layout_passes=False),
        out_type=jax.ShapeDtypeStruct((N, ENT_DIM), jnp.float32),
        scratch_types=[
            pltpu.VMEM((PER_W,), jnp.int32),
            pltpu.VMEM((CHUNK, ENT_DIM), jnp.float32),
            pltpu.VMEM((CHUNK, ENT_DIM), jnp.float32),
            pltpu.VMEM((CHUNK, ENT_DIM), jnp.float32),
            pltpu.SemaphoreType.DMA,
            pltpu.SemaphoreType.DMA,
        ],
    )
    def rel_kernel(br_hbm, rel_hbm, rhat_hbm, bid_r, rw0, rw1, nrm, sem0, sem1):
        wid = lax.axis_index("s") * NC + lax.axis_index("c")
        base = wid * PER_W
        pltpu.sync_copy(br_hbm.at[pl.ds(base, PER_W)], bid_r)
        bufs = [(rw0, sem0), (rw1, sem1)]

        def fire(c, parity):
            rw, sem = bufs[parity]
            def vblock(vb, _):
                vr = bid_r[pl.ds(c * CHUNK + vb * LANES, LANES)]
                for j in range(LANES):
                    pltpu.async_copy(rel_hbm.at[vr[j]], rw.at[vb * LANES + j], sem)
                return 0
            lax.fori_loop(0, CHUNK // LANES, vblock, 0)

        def drain(parity):
            rw, sem = bufs[parity]
            pltpu.make_async_copy(rel_hbm.at[pl.ds(0, CHUNK)], rw, sem).wait()

        def process(c, parity):
            rw, _ = bufs[parity]
            def row_body(i, _):
                r = [rw[i, pl.ds(k * LANES, LANES)] for k in range(KV)]
                inr = _inv_norm(r)
                for k in range(KV):
                    nrm[i, pl.ds(k * LANES, LANES)] = r[k] * inr
                return 0
            lax.fori_loop(0, CHUNK, row_body, 0)
            pltpu.sync_copy(nrm, rhat_hbm.at[pl.ds(base + c * CHUNK, CHUNK)])

        fire(0, 0)
        fire(1, 1)

        def pair_body(p, _):
            c0 = 2 * p
            drain(0)
            process(c0, 0)
            fire(jnp.minimum(c0 + 2, NCHUNK - 1), 0)
            drain(1)
            process(c0 + 1, 1)
            fire(jnp.minimum(c0 + 3, NCHUNK - 1), 1)
            return 0

        lax.fori_loop(0, NPAIR, pair_body, 0)
        drain(0)
        drain(1)

    return rel_kernel


def _make_main_kernel():
    mesh = plsc.VectorSubcoreMesh(core_axis_name="c", subcore_axis_name="s")

    @functools.partial(
        pl.kernel,
        mesh=mesh,
        compiler_params=pltpu.CompilerParams(needs_layout_passes=False),
        out_type=jax.ShapeDtypeStruct((NW, LANES), jnp.float32),
        scratch_types=[
            pltpu.VMEM((PER_W,), jnp.int32),
            pltpu.VMEM((PER_W,), jnp.int32),
            pltpu.VMEM((CHUNK, ENT_DIM), jnp.float32),
            pltpu.VMEM((CHUNK, ENT_DIM), jnp.float32),
            pltpu.VMEM((CHUNK, ENT_DIM), jnp.float32),
            pltpu.VMEM((CHUNK, ENT_DIM), jnp.float32),
            pltpu.VMEM((CHUNK, ENT_DIM), jnp.float32),
            pltpu.VMEM((CHUNK, ENT_DIM), jnp.float32),
            pltpu.VMEM((LANES,), jnp.float32),
            pltpu.SemaphoreType.DMA,
            pltpu.SemaphoreType.DMA,
        ],
    )
    def sc_kernel(bh_hbm, bt_hbm, ent_hbm, rhat_hbm, out_hbm,
                  bid_h, bid_t,
                  rh0, rr0, rt0, rh1, rr1, rt1, part_v, sem0, sem1):
        wid = lax.axis_index("s") * NC + lax.axis_index("c")
        base = wid * PER_W
        pltpu.sync_copy(bh_hbm.at[pl.ds(base, PER_W)], bid_h)
        pltpu.sync_copy(bt_hbm.at[pl.ds(base, PER_W)], bid_t)

        bufs = [(rh0, rr0, rt0, sem0), (rh1, rr1, rt1, sem1)]

        def fire(c, parity):
            rh, rr, rt, sem = bufs[parity]
            pltpu.async_copy(rhat_hbm.at[pl.ds(base + c * CHUNK, CHUNK)], rr, sem)
            def vblock(vb, _):
                cb = c * CHUNK + vb * LANES
                vh = bid_h[pl.ds(cb, LANES)]
                vt = bid_t[pl.ds(cb, LANES)]
                for j in range(LANES):
                    row = vb * LANES + j
                    pltpu.async_copy(ent_hbm.at[vh[j]], rh.at[row], sem)
                    pltpu.async_copy(ent_hbm.at[vt[j]], rt.at[row], sem)
                return 0
            lax.fori_loop(0, CHUNK // LANES, vblock, 0)

        def drain(parity):
            rh, rr, rt, sem = bufs[parity]
            pltpu.make_async_copy(ent_hbm.at[pl.ds(0, CHUNK)], rh, sem).wait()
            pltpu.make_async_copy(ent_hbm.at[pl.ds(0, CHUNK)], rr, sem).wait()
            pltpu.make_async_copy(ent_hbm.at[pl.ds(0, CHUNK)], rt, sem).wait()

        def compute(parity, loss):
            rh, rr, rt, _ = bufs[parity]

            def group_body(g, loss_acc):
                gb = g * (NEG_RATIO + 1)
                gacc = jnp.zeros((LANES,), jnp.float32)
                for j in range(NEG_RATIO + 1):
                    row = gb + j
                    w = jnp.float32(1.0 if j == 0 else -1.0 / NEG_RATIO)
                    h = [rh[row, pl.ds(k * LANES, LANES)] for k in range(KV)]
                    r = [rr[row, pl.ds(k * LANES, LANES)] for k in range(KV)]
                    t = [rt[row, pl.ds(k * LANES, LANES)] for k in range(KV)]
                    inh = _inv_norm(h)
                    int_ = _inv_norm(t)
                    acc = jnp.abs(h[0] * inh + r[0] - t[0] * int_)
                    for k in range(1, KV):
                        acc = acc + jnp.abs(h[k] * inh + r[k] - t[k] * int_)
                    gacc = gacc + w * acc
                term = jnp.sum(gacc) + jnp.float32(MARGIN)
                return loss_acc + jnp.maximum(term, 0.0)

            return lax.fori_loop(0, GROUPS_PER_CHUNK, group_body, loss)

        fire(0, 0)
        fire(1, 1)

        def pair_body(p, loss):
            c0 = 2 * p
            drain(0)
            loss = compute(0, loss)
            fire(jnp.minimum(c0 + 2, NCHUNK - 1), 0)
            drain(1)
            loss = compute(1, loss)
            fire(jnp.minimum(c0 + 3, NCHUNK - 1), 1)
            return loss

        loss = lax.fori_loop(0, NPAIR, pair_body, jnp.float32(0.0))
        drain(0)
        drain(1)

        part_v[...] = jnp.full((LANES,), loss)
        pltpu.sync_copy(part_v, out_hbm.at[wid])

    return sc_kernel


_TC_TRANSPOSE = _make_tc_transpose()
_REL_KERNEL = _make_rel_kernel()
_MAIN_KERNEL = _make_main_kernel()


def kernel(batch_h, batch_r, batch_t, batch_y, ent_table, rel_table):
    del batch_y
    rhat = _REL_KERNEL(batch_r.astype(jnp.int32), rel_table)
    ent_rows = _TC_TRANSPOSE(ent_table.T)
    parts = _MAIN_KERNEL(batch_h.astype(jnp.int32), batch_t.astype(jnp.int32),
                         ent_rows, rhat)
    return jnp.sum(parts[:, 0])


# speedup vs baseline: 2.2873x; 1.0850x over previous
import functools

import jax
import jax.numpy as jnp
from jax import lax
from jax.experimental import pallas as pl
from jax.experimental.pallas import tpu as pltpu
from jax.experimental.pallas import tpu_sc as plsc

NUM_ENTS = 1000000
ENT_DIM = 64
NEG_RATIO = 7
MARGIN = 1.0
N = 32768

NC = 2
NS = 16
NW = NC * NS
LANES = 16
PER_W = N // NW
CHUNK = 64
NCHUNK = PER_W // CHUNK
NPAIR = NCHUNK // 2
GROUPS_PER_CHUNK = CHUNK // (NEG_RATIO + 1)
KV = ENT_DIM // LANES


def _rsqrt_nr(s):
    i = lax.bitcast_convert_type(s, jnp.int32)
    i = jnp.int32(0x5F3759DF) - (i >> 1)
    y = lax.bitcast_convert_type(i, jnp.float32)
    half_s = s * 0.5
    for _ in range(3):
        y = y * (1.5 - half_s * y * y)
    return y


def _inv_norm(vregs):
    ss = vregs[0] * vregs[0]
    for k in range(1, len(vregs)):
        ss = ss + vregs[k] * vregs[k]
    return _rsqrt_nr(jnp.full((LANES,), jnp.maximum(jnp.sum(ss), 1e-24)))


TC_BLOCK = 8192
HALF_BLOCK = TC_BLOCK // 2
N_TC_STEPS = -(-NUM_ENTS // TC_BLOCK)
PAIR_ROWS = N_TC_STEPS * HALF_BLOCK


def _make_tc_transpose():

    def body(x_ref, o_ref):
        xt = x_ref[...].T
        o_ref[...] = jnp.concatenate(
            [xt[:HALF_BLOCK], xt[HALF_BLOCK:]], axis=1)

    return pl.pallas_call(
        body,
        grid=(N_TC_STEPS,),
        in_specs=[pl.BlockSpec((ENT_DIM, TC_BLOCK), lambda i: (0, i))],
        out_specs=pl.BlockSpec((HALF_BLOCK, 2 * ENT_DIM), lambda i: (i, 0)),
        out_shape=jax.ShapeDtypeStruct((PAIR_ROWS, 2 * ENT_DIM), jnp.float32),
    )


def _make_rel_kernel():
    mesh = plsc.VectorSubcoreMesh(core_axis_name="c", subcore_axis_name="s")

    @functools.partial(
        pl.kernel,
        mesh=mesh,
        compiler_params=pltpu.CompilerParams(needs_layout_passes=False),
        out_type=jax.ShapeDtypeStruct((N, ENT_DIM), jnp.float32),
        scratch_types=[
            pltpu.VMEM((PER_W,), jnp.int32),
            pltpu.VMEM((CHUNK, ENT_DIM), jnp.float32),
            pltpu.VMEM((CHUNK, ENT_DIM), jnp.float32),
            pltpu.VMEM((CHUNK, ENT_DIM), jnp.float32),
            pltpu.SemaphoreType.DMA,
            pltpu.SemaphoreType.DMA,
        ],
    )
    def rel_kernel(br_hbm, rel_hbm, rhat_hbm, bid_r, rw0, rw1, nrm, sem0, sem1):
        wid = lax.axis_index("s") * NC + lax.axis_index("c")
        base = wid * PER_W
        pltpu.sync_copy(br_hbm.at[pl.ds(base, PER_W)], bid_r)
        bufs = [(rw0, sem0), (rw1, sem1)]

        def fire(c, parity):
            rw, sem = bufs[parity]
            def vblock(vb, _):
                vr = bid_r[pl.ds(c * CHUNK + vb * LANES, LANES)]
                for j in range(LANES):
                    pltpu.async_copy(rel_hbm.at[vr[j]], rw.at[vb * LANES + j], sem)
                return 0
            lax.fori_loop(0, CHUNK // LANES, vblock, 0)

        def drain(parity):
            rw, sem = bufs[parity]
            pltpu.make_async_copy(rel_hbm.at[pl.ds(0, CHUNK)], rw, sem).wait()

        def process(c, parity):
            rw, _ = bufs[parity]
            def row_body(i, _):
                r = [rw[i, pl.ds(k * LANES, LANES)] for k in range(KV)]
                inr = _inv_norm(r)
                for k in range(KV):
                    nrm[i, pl.ds(k * LANES, LANES)] = r[k] * inr
                return 0
            lax.fori_loop(0, CHUNK, row_body, 0)
            pltpu.sync_copy(nrm, rhat_hbm.at[pl.ds(base + c * CHUNK, CHUNK)])

        fire(0, 0)
        fire(1, 1)

        def pair_body(p, _):
            c0 = 2 * p
            drain(0)
            process(c0, 0)
            fire(jnp.minimum(c0 + 2, NCHUNK - 1), 0)
            drain(1)
            process(c0 + 1, 1)
            fire(jnp.minimum(c0 + 3, NCHUNK - 1), 1)
            return 0

        lax.fori_loop(0, NPAIR, pair_body, 0)
        drain(0)
        drain(1)

    return rel_kernel


def _make_main_kernel():
    mesh = plsc.VectorSubcoreMesh(core_axis_name="c", subcore_axis_name="s")

    @functools.partial(
        pl.kernel,
        mesh=mesh,
        compiler_params=pltpu.CompilerParams(needs_layout_passes=False),
        out_type=jax.ShapeDtypeStruct((NW, LANES), jnp.float32),
        scratch_types=[
            pltpu.VMEM((PER_W,), jnp.int32),
            pltpu.VMEM((PER_W,), jnp.int32),
            pltpu.VMEM((PER_W,), jnp.int32),
            pltpu.VMEM((PER_W,), jnp.int32),
            pltpu.VMEM((CHUNK, 2 * ENT_DIM), jnp.float32),
            pltpu.VMEM((CHUNK, ENT_DIM), jnp.float32),
            pltpu.VMEM((CHUNK, 2 * ENT_DIM), jnp.float32),
            pltpu.VMEM((CHUNK, 2 * ENT_DIM), jnp.float32),
            pltpu.VMEM((CHUNK, ENT_DIM), jnp.float32),
            pltpu.VMEM((CHUNK, 2 * ENT_DIM), jnp.float32),
            pltpu.VMEM((LANES,), jnp.float32),
            pltpu.SemaphoreType.DMA,
            pltpu.SemaphoreType.DMA,
        ],
    )
    def sc_kernel(bh_hbm, bt_hbm, ent_hbm, rhat_hbm, out_hbm,
                  bid_h, bid_t, idx_h, idx_t,
                  rh0, rr0, rt0, rh1, rr1, rt1, part_v, sem0, sem1):
        wid = lax.axis_index("s") * NC + lax.axis_index("c")
        base = wid * PER_W
        pltpu.sync_copy(bh_hbm.at[pl.ds(base, PER_W)], bid_h)
        pltpu.sync_copy(bt_hbm.at[pl.ds(base, PER_W)], bid_t)
        for v in range(PER_W // LANES):
            sl = pl.ds(v * LANES, LANES)
            vh = bid_h[sl]
            vt = bid_t[sl]
            idx_h[sl] = ((vh >> 13) << 12) + (vh & (HALF_BLOCK - 1))
            idx_t[sl] = ((vt >> 13) << 12) + (vt & (HALF_BLOCK - 1))

        bufs = [(rh0, rr0, rt0, sem0), (rh1, rr1, rt1, sem1)]

        def fire(c, parity):
            rh, rr, rt, sem = bufs[parity]
            csl = pl.ds(c * CHUNK, CHUNK)
            pltpu.async_copy(rhat_hbm.at[pl.ds(base + c * CHUNK, CHUNK)], rr, sem)
            pltpu.async_copy(ent_hbm.at[idx_h.at[csl]], rh, sem)
            pltpu.async_copy(ent_hbm.at[idx_t.at[csl]], rt, sem)

        def drain(parity):
            rh, rr, rt, sem = bufs[parity]
            pltpu.make_async_copy(ent_hbm.at[pl.ds(0, CHUNK)], rh, sem).wait()
            pltpu.make_async_copy(rhat_hbm.at[pl.ds(0, CHUNK)], rr, sem).wait()
            pltpu.make_async_copy(ent_hbm.at[pl.ds(0, CHUNK)], rt, sem).wait()

        def compute(c, parity, loss):
            rh, rr, rt, _ = bufs[parity]

            def group_body(g, loss_acc, c=c):
                gb = g * (NEG_RATIO + 1)
                vh = bid_h[pl.ds(c * CHUNK + gb, LANES)]
                vt = bid_t[pl.ds(c * CHUNK + gb, LANES)]
                gacc = jnp.zeros((LANES,), jnp.float32)
                for j in range(NEG_RATIO + 1):
                    row = gb + j
                    w = jnp.float32(1.0 if j == 0 else -1.0 / NEG_RATIO)
                    oh = ((vh[j] >> 12) & 1) * ENT_DIM
                    ot = ((vt[j] >> 12) & 1) * ENT_DIM
                    h = [rh[row, pl.ds(oh + k * LANES, LANES)] for k in range(KV)]
                    r = [rr[row, pl.ds(k * LANES, LANES)] for k in range(KV)]
                    t = [rt[row, pl.ds(ot + k * LANES, LANES)] for k in range(KV)]
                    inh = _inv_norm(h)
                    int_ = _inv_norm(t)
                    acc = jnp.abs(h[0] * inh + r[0] - t[0] * int_)
                    for k in range(1, KV):
                        acc = acc + jnp.abs(h[k] * inh + r[k] - t[k] * int_)
                    gacc = gacc + w * acc
                term = jnp.sum(gacc) + jnp.float32(MARGIN)
                return loss_acc + jnp.maximum(term, 0.0)

            return lax.fori_loop(0, GROUPS_PER_CHUNK, group_body, loss)

        fire(0, 0)
        fire(1, 1)

        def pair_body(p, loss):
            c0 = 2 * p
            drain(0)
            loss = compute(c0, 0, loss)
            fire(jnp.minimum(c0 + 2, NCHUNK - 1), 0)
            drain(1)
            loss = compute(c0 + 1, 1, loss)
            fire(jnp.minimum(c0 + 3, NCHUNK - 1), 1)
            return loss

        loss = lax.fori_loop(0, NPAIR, pair_body, jnp.float32(0.0))
        drain(0)
        drain(1)

        part_v[...] = jnp.full((LANES,), loss)
        pltpu.sync_copy(part_v, out_hbm.at[wid])

    return sc_kernel


_TC_TRANSPOSE = _make_tc_transpose()
_REL_KERNEL = _make_rel_kernel()
_MAIN_KERNEL = _make_main_kernel()


def kernel(batch_h, batch_r, batch_t, batch_y, ent_table, rel_table):
    del batch_y
    rhat = _REL_KERNEL(batch_r.astype(jnp.int32), rel_table)
    ent_rows = _TC_TRANSPOSE(ent_table.T)
    parts = _MAIN_KERNEL(batch_h.astype(jnp.int32), batch_t.astype(jnp.int32),
                         ent_rows, rhat)
    return jnp.sum(parts[:, 0])


# speedup vs baseline: 2.6630x; 1.1643x over previous
import functools

import jax
import jax.numpy as jnp
from jax import lax
from jax.experimental import pallas as pl
from jax.experimental.pallas import tpu as pltpu
from jax.experimental.pallas import tpu_sc as plsc

NUM_ENTS = 1000000
ENT_DIM = 64
NEG_RATIO = 7
MARGIN = 1.0
N = 32768

NC = 2
NS = 16
NW = NC * NS
LANES = 16
PER_W = N // NW
CHUNK = 64
NCHUNK = PER_W // CHUNK
NPAIR = NCHUNK // 2
GROUPS_PER_CHUNK = CHUNK // (NEG_RATIO + 1)
KV = ENT_DIM // LANES


def _rsqrt_nr(s):
    i = lax.bitcast_convert_type(s, jnp.int32)
    i = jnp.int32(0x5F3759DF) - (i >> 1)
    y = lax.bitcast_convert_type(i, jnp.float32)
    half_s = s * 0.5
    for _ in range(3):
        y = y * (1.5 - half_s * y * y)
    return y


def _inv_norm(vregs):
    ss = vregs[0] * vregs[0]
    for k in range(1, len(vregs)):
        ss = ss + vregs[k] * vregs[k]
    return _rsqrt_nr(jnp.full((LANES,), jnp.maximum(jnp.sum(ss), 1e-24)))


TC_BLOCK = 32768
HALF_BLOCK = TC_BLOCK // 2
N_TC_STEPS = -(-NUM_ENTS // TC_BLOCK)
PAIR_ROWS = N_TC_STEPS * HALF_BLOCK


def _make_tc_transpose():

    def body(x_ref, o_ref):
        xt = x_ref[...].T
        o_ref[...] = jnp.concatenate(
            [xt[:HALF_BLOCK], xt[HALF_BLOCK:]], axis=1)

    return pl.pallas_call(
        body,
        grid=(N_TC_STEPS,),
        in_specs=[pl.BlockSpec((ENT_DIM, TC_BLOCK), lambda i: (0, i))],
        out_specs=pl.BlockSpec((HALF_BLOCK, 2 * ENT_DIM), lambda i: (i, 0)),
        out_shape=jax.ShapeDtypeStruct((PAIR_ROWS, 2 * ENT_DIM), jnp.float32),
    )


def _make_rel_kernel():
    mesh = plsc.VectorSubcoreMesh(core_axis_name="c", subcore_axis_name="s")

    @functools.partial(
        pl.kernel,
        mesh=mesh,
        compiler_params=pltpu.CompilerParams(needs_layout_passes=False),
        out_type=jax.ShapeDtypeStruct((N, ENT_DIM), jnp.float32),
        scratch_types=[
            pltpu.VMEM((PER_W,), jnp.int32),
            pltpu.VMEM((CHUNK, ENT_DIM), jnp.float32),
            pltpu.VMEM((CHUNK, ENT_DIM), jnp.float32),
            pltpu.VMEM((CHUNK, ENT_DIM), jnp.float32),
            pltpu.SemaphoreType.DMA,
            pltpu.SemaphoreType.DMA,
        ],
    )
    def rel_kernel(br_hbm, rel_hbm, rhat_hbm, bid_r, rw0, rw1, nrm, sem0, sem1):
        wid = lax.axis_index("s") * NC + lax.axis_index("c")
        base = wid * PER_W
        pltpu.sync_copy(br_hbm.at[pl.ds(base, PER_W)], bid_r)
        bufs = [(rw0, sem0), (rw1, sem1)]

        def fire(c, parity):
            rw, sem = bufs[parity]
            def vblock(vb, _):
                vr = bid_r[pl.ds(c * CHUNK + vb * LANES, LANES)]
                for j in range(LANES):
                    pltpu.async_copy(rel_hbm.at[vr[j]], rw.at[vb * LANES + j], sem)
                return 0
            lax.fori_loop(0, CHUNK // LANES, vblock, 0)

        def drain(parity):
            rw, sem = bufs[parity]
            pltpu.make_async_copy(rel_hbm.at[pl.ds(0, CHUNK)], rw, sem).wait()

        def process(c, parity):
            rw, _ = bufs[parity]
            def row_body(i, _):
                r = [rw[i, pl.ds(k * LANES, LANES)] for k in range(KV)]
                inr = _inv_norm(r)
                for k in range(KV):
                    nrm[i, pl.ds(k * LANES, LANES)] = r[k] * inr
                return 0
            lax.fori_loop(0, CHUNK, row_body, 0)
            pltpu.sync_copy(nrm, rhat_hbm.at[pl.ds(base + c * CHUNK, CHUNK)])

        fire(0, 0)
        fire(1, 1)

        def pair_body(p, _):
            c0 = 2 * p
            drain(0)
            process(c0, 0)
            fire(jnp.minimum(c0 + 2, NCHUNK - 1), 0)
            drain(1)
            process(c0 + 1, 1)
            fire(jnp.minimum(c0 + 3, NCHUNK - 1), 1)
            return 0

        lax.fori_loop(0, NPAIR, pair_body, 0)
        drain(0)
        drain(1)

    return rel_kernel


def _make_main_kernel():
    mesh = plsc.VectorSubcoreMesh(core_axis_name="c", subcore_axis_name="s")

    @functools.partial(
        pl.kernel,
        mesh=mesh,
        compiler_params=pltpu.CompilerParams(needs_layout_passes=False),
        out_type=jax.ShapeDtypeStruct((NW, LANES), jnp.float32),
        scratch_types=[
            pltpu.VMEM((PER_W,), jnp.int32),
            pltpu.VMEM((PER_W,), jnp.int32),
            pltpu.VMEM((PER_W,), jnp.int32),
            pltpu.VMEM((PER_W,), jnp.int32),
            pltpu.VMEM((CHUNK, 2 * ENT_DIM), jnp.float32),
            pltpu.VMEM((CHUNK, ENT_DIM), jnp.float32),
            pltpu.VMEM((CHUNK, 2 * ENT_DIM), jnp.float32),
            pltpu.VMEM((CHUNK, 2 * ENT_DIM), jnp.float32),
            pltpu.VMEM((CHUNK, ENT_DIM), jnp.float32),
            pltpu.VMEM((CHUNK, 2 * ENT_DIM), jnp.float32),
            pltpu.VMEM((LANES,), jnp.float32),
            pltpu.SemaphoreType.DMA,
            pltpu.SemaphoreType.DMA,
        ],
    )
    def sc_kernel(bh_hbm, bt_hbm, ent_hbm, rhat_hbm, out_hbm,
                  bid_h, bid_t, idx_h, idx_t,
                  rh0, rr0, rt0, rh1, rr1, rt1, part_v, sem0, sem1):
        wid = lax.axis_index("s") * NC + lax.axis_index("c")
        base = wid * PER_W
        pltpu.sync_copy(bh_hbm.at[pl.ds(base, PER_W)], bid_h)
        pltpu.sync_copy(bt_hbm.at[pl.ds(base, PER_W)], bid_t)
        for v in range(PER_W // LANES):
            sl = pl.ds(v * LANES, LANES)
            vh = bid_h[sl]
            vt = bid_t[sl]
            idx_h[sl] = ((vh >> 15) << 14) + (vh & (HALF_BLOCK - 1))
            idx_t[sl] = ((vt >> 15) << 14) + (vt & (HALF_BLOCK - 1))

        bufs = [(rh0, rr0, rt0, sem0), (rh1, rr1, rt1, sem1)]

        def fire(c, parity):
            rh, rr, rt, sem = bufs[parity]
            csl = pl.ds(c * CHUNK, CHUNK)
            pltpu.async_copy(rhat_hbm.at[pl.ds(base + c * CHUNK, CHUNK)], rr, sem)
            pltpu.async_copy(ent_hbm.at[idx_h.at[csl]], rh, sem)
            pltpu.async_copy(ent_hbm.at[idx_t.at[csl]], rt, sem)

        def drain(parity):
            rh, rr, rt, sem = bufs[parity]
            pltpu.make_async_copy(ent_hbm.at[pl.ds(0, CHUNK)], rh, sem).wait()
            pltpu.make_async_copy(rhat_hbm.at[pl.ds(0, CHUNK)], rr, sem).wait()
            pltpu.make_async_copy(ent_hbm.at[pl.ds(0, CHUNK)], rt, sem).wait()

        def compute(c, parity, loss):
            rh, rr, rt, _ = bufs[parity]

            def group_body(g, loss_acc, c=c):
                gb = g * (NEG_RATIO + 1)
                vh = bid_h[pl.ds(c * CHUNK + gb, LANES)]
                vt = bid_t[pl.ds(c * CHUNK + gb, LANES)]
                gacc = jnp.zeros((LANES,), jnp.float32)
                for j in range(NEG_RATIO + 1):
                    row = gb + j
                    w = jnp.float32(1.0 if j == 0 else -1.0 / NEG_RATIO)
                    oh = ((vh[j] >> 14) & 1) * ENT_DIM
                    ot = ((vt[j] >> 14) & 1) * ENT_DIM
                    h = [rh[row, pl.ds(oh + k * LANES, LANES)] for k in range(KV)]
                    r = [rr[row, pl.ds(k * LANES, LANES)] for k in range(KV)]
                    t = [rt[row, pl.ds(ot + k * LANES, LANES)] for k in range(KV)]
                    inh = _inv_norm(h)
                    int_ = _inv_norm(t)
                    acc = jnp.abs(h[0] * inh + r[0] - t[0] * int_)
                    for k in range(1, KV):
                        acc = acc + jnp.abs(h[k] * inh + r[k] - t[k] * int_)
                    gacc = gacc + w * acc
                term = jnp.sum(gacc) + jnp.float32(MARGIN)
                return loss_acc + jnp.maximum(term, 0.0)

            return lax.fori_loop(0, GROUPS_PER_CHUNK, group_body, loss)

        fire(0, 0)
        fire(1, 1)

        def pair_body(p, loss):
            c0 = 2 * p
            drain(0)
            loss = compute(c0, 0, loss)
            fire(jnp.minimum(c0 + 2, NCHUNK - 1), 0)
            drain(1)
            loss = compute(c0 + 1, 1, loss)
            fire(jnp.minimum(c0 + 3, NCHUNK - 1), 1)
            return loss

        loss = lax.fori_loop(0, NPAIR, pair_body, jnp.float32(0.0))
        drain(0)
        drain(1)

        part_v[...] = jnp.full((LANES,), loss)
        pltpu.sync_copy(part_v, out_hbm.at[wid])

    return sc_kernel


_TC_TRANSPOSE = _make_tc_transpose()
_REL_KERNEL = _make_rel_kernel()
_MAIN_KERNEL = _make_main_kernel()


def kernel(batch_h, batch_r, batch_t, batch_y, ent_table, rel_table):
    del batch_y
    rhat = _REL_KERNEL(batch_r.astype(jnp.int32), rel_table)
    ent_rows = _TC_TRANSPOSE(ent_table.T)
    parts = _MAIN_KERNEL(batch_h.astype(jnp.int32), batch_t.astype(jnp.int32),
                         ent_rows, rhat)
    return jnp.sum(parts[:, 0])
